# baseline (speedup 1.0000x reference)
import functools

import jax
import jax.numpy as jnp
from jax import lax
from jax.experimental import pallas as pl
from jax.experimental.pallas import tpu as pltpu
from jax.experimental.pallas import tpu_sc as plsc

NC, NS, L = 2, 16, 16
NW = NC * NS

EPS = 1e-6


def _make_score_kernel(T, N, D, E):
    TOT = T * 2 * E
    PER_W = TOT // NW
    C = 80
    NCHUNK = PER_W // C
    assert PER_W * NW == TOT and NCHUNK * C == PER_W and D == 16 * L

    mesh = plsc.VectorSubcoreMesh(core_axis_name="c", subcore_axis_name="s")

    @functools.partial(
        pl.kernel,
        mesh=mesh,
        out_type=jax.ShapeDtypeStruct((TOT,), jnp.float32),
        compiler_params=pltpu.CompilerParams(use_tc_tiling_on_sc=False,
                                             needs_layout_passes=False),
        scratch_types=[
            pltpu.VMEM((C,), jnp.int32),
            pltpu.VMEM((C,), jnp.int32),
            pltpu.VMEM((C, D), jnp.float32),
            pltpu.VMEM((C, D), jnp.float32),
            pltpu.VMEM((C,), jnp.int32),
            pltpu.VMEM((C,), jnp.int32),
            pltpu.VMEM((C, D), jnp.float32),
            pltpu.VMEM((C, D), jnp.float32),
            pltpu.VMEM((C,), jnp.float32),
            pltpu.SemaphoreType.DMA,
            pltpu.SemaphoreType.DMA,
            pltpu.SemaphoreType.DMA,
            pltpu.SemaphoreType.DMA,
        ],
    )
    def score_kernel(z_hbm, src_hbm, dst_hbm, out_hbm,
                     sidx0, didx0, srows0, drows0,
                     sidx1, didx1, srows1, drows1,
                     scv, sem0a, sem0b, sem1a, sem1b):
        wid = lax.axis_index("s") * NC + lax.axis_index("c")
        base = wid * PER_W
        bufs = ((sidx0, didx0, srows0, drows0, sem0a, sem0b),
                (sidx1, didx1, srows1, drows1, sem1a, sem1b))

        def start(ci, b):
            sidx, didx, srows, drows, sa, sb = b
            off = base + ci * C
            pltpu.sync_copy(src_hbm.at[pl.ds(off, C)], sidx)
            pltpu.sync_copy(dst_hbm.at[pl.ds(off, C)], didx)
            pltpu.async_copy(z_hbm.at[sidx], srows, sa)
            pltpu.async_copy(z_hbm.at[didx], drows, sb)

        def waitb(b):
            sidx, didx, srows, drows, sa, sb = b
            pltpu.make_async_copy(z_hbm.at[sidx], srows, sa).wait()
            pltpu.make_async_copy(z_hbm.at[didx], drows, sb).wait()

        ii = lax.iota(jnp.int32, L)

        def compute(ci, b):
            sidx, didx, srows, drows, _, _ = b
            off = base + ci * C

            zvec = jnp.zeros((L,), jnp.int32)

            def group(g, carry2):
                base_e = g * L
                iv0 = base_e * D + ii * D
                zv = jnp.zeros((L,), jnp.float32)

                def dstep(t, carry):
                    a0, a1, a2, a3, iv = carry
                    outs = []
                    for u, acc in zip(range(4), (a0, a1, a2, a3)):
                        ivu = iv + u
                        a = plsc.load_gather(srows, [zvec, ivu])
                        bb = plsc.load_gather(drows, [zvec, ivu])
                        outs.append(acc + a * bb)
                    return (*outs, iv + 4)

                a0, a1, a2, a3, _ = lax.fori_loop(
                    0, D // 4, dstep, (zv, zv, zv, zv, iv0), unroll=4)
                acc = (a0 + a1) + (a2 + a3)
                scv[pl.ds(base_e, L)] = acc
                return carry2

            lax.fori_loop(0, C // L, group, 0)
            pltpu.sync_copy(scv, out_hbm.at[pl.ds(off, C)])

        start(0, bufs[0])

        def body(p, carry):
            c0 = 2 * p
            start(c0 + 1, bufs[1])
            waitb(bufs[0])
            compute(c0, bufs[0])

            @pl.when(c0 + 2 < NCHUNK)
            def _():
                start(c0 + 2, bufs[0])

            waitb(bufs[1])
            compute(c0 + 1, bufs[1])
            return carry

        lax.fori_loop(0, NCHUNK // 2, body, 0)

    return score_kernel


def _loss_body(T, E, p_ref, n_ref, o_ref):
    p = p_ref[...]
    n = n_ref[...]
    sp = 1.0 / (1.0 + jnp.exp(-p))
    sn = 1.0 / (1.0 + jnp.exp(-n))
    tp = jnp.log(sp + EPS)
    tn = jnp.log(1.0 - sn + EPS)
    o_ref[0, 0] = -(jnp.sum(tp) + jnp.sum(tn)) / (T * E)


def kernel(ps, ns, zs):
    T, N, D = zs.shape
    E = ps.shape[2]

    zf = zs.reshape(T * N, D)
    offs = (jnp.arange(T, dtype=jnp.int32) * N)[:, None, None]
    src = (jnp.stack([ps[:, 0, :], ns[:, 0, :]], axis=1).astype(jnp.int32)
           + offs).reshape(-1)
    dst = (jnp.stack([ps[:, 1, :], ns[:, 1, :]], axis=1).astype(jnp.int32)
           + offs).reshape(-1)

    scores = _make_score_kernel(T, N, D, E)(zf, src, dst)
    sc4 = scores.reshape(T, 2, E)
    pos = sc4[:, 0, :]
    neg = sc4[:, 1, :]

    loss = pl.pallas_call(
        functools.partial(_loss_body, T, E),
        out_shape=jax.ShapeDtypeStruct((1, 1), jnp.float32),
        in_specs=[
            pl.BlockSpec(memory_space=pltpu.VMEM),
            pl.BlockSpec(memory_space=pltpu.VMEM),
        ],
        out_specs=pl.BlockSpec(memory_space=pltpu.SMEM),
    )(pos, neg)
    return loss.reshape(1)


# speedup vs baseline: 5.0168x; 5.0168x over previous
import functools

import jax
import jax.numpy as jnp
from jax import lax
from jax.experimental import pallas as pl
from jax.experimental.pallas import tpu as pltpu
from jax.experimental.pallas import tpu_sc as plsc

NC, NS, L = 2, 16, 16
NW = NC * NS

EPS = 1e-6


def _make_score_kernel(T, N, D, E):
    TOT = T * 2 * E
    PER_W = TOT // NW
    C = 80
    NCHUNK = PER_W // C
    assert PER_W * NW == TOT and NCHUNK * C == PER_W and D == 16 * L

    mesh = plsc.VectorSubcoreMesh(core_axis_name="c", subcore_axis_name="s")

    @functools.partial(
        pl.kernel,
        mesh=mesh,
        out_type=jax.ShapeDtypeStruct((TOT,), jnp.float32),
        compiler_params=pltpu.CompilerParams(use_tc_tiling_on_sc=False,
                                             needs_layout_passes=False),
        scratch_types=[
            pltpu.VMEM((C,), jnp.int32),
            pltpu.VMEM((C,), jnp.int32),
            pltpu.VMEM((C, D), jnp.float32),
            pltpu.VMEM((C, D), jnp.float32),
            pltpu.VMEM((C,), jnp.int32),
            pltpu.VMEM((C,), jnp.int32),
            pltpu.VMEM((C, D), jnp.float32),
            pltpu.VMEM((C, D), jnp.float32),
            pltpu.VMEM((C,), jnp.float32),
            pltpu.SemaphoreType.DMA,
            pltpu.SemaphoreType.DMA,
            pltpu.SemaphoreType.DMA,
            pltpu.SemaphoreType.DMA,
        ],
    )
    def score_kernel(z_hbm, src_hbm, dst_hbm, out_hbm,
                     sidx0, didx0, srows0, drows0,
                     sidx1, didx1, srows1, drows1,
                     scv, sem0a, sem0b, sem1a, sem1b):
        wid = lax.axis_index("s") * NC + lax.axis_index("c")
        base = wid * PER_W
        bufs = ((sidx0, didx0, srows0, drows0, sem0a, sem0b),
                (sidx1, didx1, srows1, drows1, sem1a, sem1b))

        def start(ci, b):
            sidx, didx, srows, drows, sa, sb = b
            off = base + ci * C
            pltpu.sync_copy(src_hbm.at[pl.ds(off, C)], sidx)
            pltpu.sync_copy(dst_hbm.at[pl.ds(off, C)], didx)
            pltpu.async_copy(z_hbm.at[sidx], srows, sa)
            pltpu.async_copy(z_hbm.at[didx], drows, sb)

        def waitb(b):
            sidx, didx, srows, drows, sa, sb = b
            pltpu.make_async_copy(z_hbm.at[sidx], srows, sa).wait()
            pltpu.make_async_copy(z_hbm.at[didx], drows, sb).wait()

        ii = lax.iota(jnp.int32, L)

        def compute(ci, b):
            sidx, didx, srows, drows, _, _ = b
            off = base + ci * C

            def group(g, carry2):
                base_e = g * L

                def edge(j, svec):
                    e = base_e + j
                    parts = []
                    for m in range(4):
                        p = (srows[e, pl.ds(m * 64, L)]
                             * drows[e, pl.ds(m * 64, L)])
                        for k in range(1, 4):
                            col = m * 64 + k * L
                            p = p + (srows[e, pl.ds(col, L)]
                                     * drows[e, pl.ds(col, L)])
                        parts.append(p)
                    acc = (parts[0] + parts[1]) + (parts[2] + parts[3])
                    return jnp.where(ii == j, jnp.sum(acc), svec)

                svec = lax.fori_loop(0, L, edge, jnp.zeros((L,), jnp.float32),
                                     unroll=4)
                scv[pl.ds(base_e, L)] = svec
                return carry2

            lax.fori_loop(0, C // L, group, 0)
            pltpu.sync_copy(scv, out_hbm.at[pl.ds(off, C)])

        start(0, bufs[0])

        def body(p, carry):
            c0 = 2 * p
            start(c0 + 1, bufs[1])
            waitb(bufs[0])
            compute(c0, bufs[0])

            @pl.when(c0 + 2 < NCHUNK)
            def _():
                start(c0 + 2, bufs[0])

            waitb(bufs[1])
            compute(c0 + 1, bufs[1])
            return carry

        lax.fori_loop(0, NCHUNK // 2, body, 0)

    return score_kernel


def _loss_body(T, E, p_ref, n_ref, o_ref):
    p = p_ref[...]
    n = n_ref[...]
    sp = 1.0 / (1.0 + jnp.exp(-p))
    sn = 1.0 / (1.0 + jnp.exp(-n))
    tp = jnp.log(sp + EPS)
    tn = jnp.log(1.0 - sn + EPS)
    o_ref[0, 0] = -(jnp.sum(tp) + jnp.sum(tn)) / (T * E)


def kernel(ps, ns, zs):
    T, N, D = zs.shape
    E = ps.shape[2]

    zf = zs.reshape(T * N, D)
    offs = (jnp.arange(T, dtype=jnp.int32) * N)[:, None, None]
    src = (jnp.stack([ps[:, 0, :], ns[:, 0, :]], axis=1).astype(jnp.int32)
           + offs).reshape(-1)
    dst = (jnp.stack([ps[:, 1, :], ns[:, 1, :]], axis=1).astype(jnp.int32)
           + offs).reshape(-1)

    scores = _make_score_kernel(T, N, D, E)(zf, src, dst)
    sc4 = scores.reshape(T, 2, E)
    pos = sc4[:, 0, :]
    neg = sc4[:, 1, :]

    loss = pl.pallas_call(
        functools.partial(_loss_body, T, E),
        out_shape=jax.ShapeDtypeStruct((1, 1), jnp.float32),
        in_specs=[
            pl.BlockSpec(memory_space=pltpu.VMEM),
            pl.BlockSpec(memory_space=pltpu.VMEM),
        ],
        out_specs=pl.BlockSpec(memory_space=pltpu.SMEM),
    )(pos, neg)
    return loss.reshape(1)


# speedup vs baseline: 9.8488x; 1.9632x over previous
import functools

import jax
import jax.numpy as jnp
from jax import lax
from jax.experimental import pallas as pl
from jax.experimental.pallas import tpu as pltpu
from jax.experimental.pallas import tpu_sc as plsc

NC, NS, L = 2, 16, 16
NW = NC * NS

EPS = 1e-6


def _make_score_kernel(T, N, D, E):
    TOT = T * 2 * E
    PER_W = TOT // NW
    C = 80
    NCHUNK = PER_W // C
    assert PER_W * NW == TOT and NCHUNK * C == PER_W and D == 16 * L

    mesh = plsc.VectorSubcoreMesh(core_axis_name="c", subcore_axis_name="s")

    @functools.partial(
        pl.kernel,
        mesh=mesh,
        out_type=jax.ShapeDtypeStruct((TOT,), jnp.float32),
        compiler_params=pltpu.CompilerParams(use_tc_tiling_on_sc=False,
                                             needs_layout_passes=False),
        scratch_types=[
            pltpu.VMEM((C,), jnp.int32),
            pltpu.VMEM((C,), jnp.int32),
            pltpu.VMEM((C, D), jnp.bfloat16),
            pltpu.VMEM((C, D), jnp.bfloat16),
            pltpu.VMEM((C,), jnp.int32),
            pltpu.VMEM((C,), jnp.int32),
            pltpu.VMEM((C, D), jnp.bfloat16),
            pltpu.VMEM((C, D), jnp.bfloat16),
            pltpu.VMEM((C,), jnp.float32),
            pltpu.SemaphoreType.DMA,
            pltpu.SemaphoreType.DMA,
            pltpu.SemaphoreType.DMA,
            pltpu.SemaphoreType.DMA,
        ],
    )
    def score_kernel(z_hbm, src_hbm, dst_hbm, out_hbm,
                     sidx0, didx0, srows0, drows0,
                     sidx1, didx1, srows1, drows1,
                     scv, sem0a, sem0b, sem1a, sem1b):
        wid = lax.axis_index("s") * NC + lax.axis_index("c")
        base = wid * PER_W
        bufs = ((sidx0, didx0, srows0, drows0, sem0a, sem0b),
                (sidx1, didx1, srows1, drows1, sem1a, sem1b))

        def start(ci, b):
            sidx, didx, srows, drows, sa, sb = b
            off = base + ci * C
            pltpu.sync_copy(src_hbm.at[pl.ds(off, C)], sidx)
            pltpu.sync_copy(dst_hbm.at[pl.ds(off, C)], didx)
            pltpu.async_copy(z_hbm.at[sidx], srows, sa)
            pltpu.async_copy(z_hbm.at[didx], drows, sb)

        def waitb(b):
            sidx, didx, srows, drows, sa, sb = b
            pltpu.make_async_copy(z_hbm.at[sidx], srows, sa).wait()
            pltpu.make_async_copy(z_hbm.at[didx], drows, sb).wait()

        ii = lax.iota(jnp.int32, L)

        def compute(ci, b):
            sidx, didx, srows, drows, _, _ = b
            off = base + ci * C

            def group(g, carry2):
                base_e = g * L

                def edge(j, svec):
                    e = base_e + j
                    parts = []
                    for m in range(4):
                        pr = (srows[e, pl.ds(m * 64, 2 * L)]
                              * drows[e, pl.ds(m * 64, 2 * L)])
                        pa, pb = plsc.unpack(pr,
                                             format=plsc.PackFormat.INTERLEAVED)
                        p0 = pa + pb
                        pr = (srows[e, pl.ds(m * 64 + 2 * L, 2 * L)]
                              * drows[e, pl.ds(m * 64 + 2 * L, 2 * L)])
                        pa, pb = plsc.unpack(pr,
                                             format=plsc.PackFormat.INTERLEAVED)
                        parts.append(p0 + (pa + pb))
                    acc = (parts[0] + parts[1]) + (parts[2] + parts[3])
                    return jnp.where(ii == j, jnp.sum(acc), svec)

                svec = lax.fori_loop(0, L, edge, jnp.zeros((L,), jnp.float32),
                                     unroll=4)
                scv[pl.ds(base_e, L)] = svec
                return carry2

            lax.fori_loop(0, C // L, group, 0)
            pltpu.sync_copy(scv, out_hbm.at[pl.ds(off, C)])

        start(0, bufs[0])

        def body(p, carry):
            c0 = 2 * p
            start(c0 + 1, bufs[1])
            waitb(bufs[0])
            compute(c0, bufs[0])

            @pl.when(c0 + 2 < NCHUNK)
            def _():
                start(c0 + 2, bufs[0])

            waitb(bufs[1])
            compute(c0 + 1, bufs[1])
            return carry

        lax.fori_loop(0, NCHUNK // 2, body, 0)

    return score_kernel


def _loss_body(T, E, p_ref, n_ref, o_ref):
    p = p_ref[...]
    n = n_ref[...]
    sp = 1.0 / (1.0 + jnp.exp(-p))
    sn = 1.0 / (1.0 + jnp.exp(-n))
    tp = jnp.log(sp + EPS)
    tn = jnp.log(1.0 - sn + EPS)
    o_ref[0, 0] = -(jnp.sum(tp) + jnp.sum(tn)) / (T * E)


def kernel(ps, ns, zs):
    T, N, D = zs.shape
    E = ps.shape[2]

    zf = zs.astype(jnp.bfloat16).reshape(T * N, D)
    offs = (jnp.arange(T, dtype=jnp.int32) * N)[:, None, None]
    src = (jnp.stack([ps[:, 0, :], ns[:, 0, :]], axis=1).astype(jnp.int32)
           + offs).reshape(-1)
    dst = (jnp.stack([ps[:, 1, :], ns[:, 1, :]], axis=1).astype(jnp.int32)
           + offs).reshape(-1)

    scores = _make_score_kernel(T, N, D, E)(zf, src, dst)
    sc4 = scores.reshape(T, 2, E)
    pos = sc4[:, 0, :]
    neg = sc4[:, 1, :]

    loss = pl.pallas_call(
        functools.partial(_loss_body, T, E),
        out_shape=jax.ShapeDtypeStruct((1, 1), jnp.float32),
        in_specs=[
            pl.BlockSpec(memory_space=pltpu.VMEM),
            pl.BlockSpec(memory_space=pltpu.VMEM),
        ],
        out_specs=pl.BlockSpec(memory_space=pltpu.SMEM),
    )(pos, neg)
    return loss.reshape(1)


# speedup vs baseline: 14.0527x; 1.4268x over previous
import functools

import jax
import jax.numpy as jnp
from jax import lax
from jax.experimental import pallas as pl
from jax.experimental.pallas import tpu as pltpu
from jax.experimental.pallas import tpu_sc as plsc

NC, NS, L = 2, 16, 16
NW = NC * NS

EPS = 1e-6


def _make_score_kernel(T, N, D, E):
    TOT = T * 2 * E
    PER_W = TOT // NW
    C = 80
    NCHUNK = PER_W // C
    assert PER_W * NW == TOT and NCHUNK * C == PER_W and D == 16 * L

    mesh = plsc.VectorSubcoreMesh(core_axis_name="c", subcore_axis_name="s")

    @functools.partial(
        pl.kernel,
        mesh=mesh,
        out_type=jax.ShapeDtypeStruct((TOT,), jnp.float32),
        compiler_params=pltpu.CompilerParams(use_tc_tiling_on_sc=False,
                                             needs_layout_passes=False),
        scratch_types=[
            pltpu.VMEM((PER_W,), jnp.int32),
            pltpu.VMEM((PER_W,), jnp.int32),
            pltpu.VMEM((C, D), jnp.bfloat16),
            pltpu.VMEM((C, D), jnp.bfloat16),
            pltpu.VMEM((C, D), jnp.bfloat16),
            pltpu.VMEM((C, D), jnp.bfloat16),
            pltpu.VMEM((C,), jnp.float32),
            pltpu.SemaphoreType.DMA,
            pltpu.SemaphoreType.DMA,
            pltpu.SemaphoreType.DMA,
            pltpu.SemaphoreType.DMA,
        ],
    )
    def score_kernel(z_hbm, src_hbm, dst_hbm, out_hbm,
                     sidx_all, didx_all,
                     srows0, drows0, srows1, drows1,
                     scv, sem0a, sem0b, sem1a, sem1b):
        wid = lax.axis_index("s") * NC + lax.axis_index("c")
        base = wid * PER_W
        bufs = ((srows0, drows0, sem0a, sem0b),
                (srows1, drows1, sem1a, sem1b))

        pltpu.sync_copy(src_hbm.at[pl.ds(base, PER_W)], sidx_all)
        pltpu.sync_copy(dst_hbm.at[pl.ds(base, PER_W)], didx_all)

        def start(ci, b):
            srows, drows, sa, sb = b
            cb = ci * C
            pltpu.async_copy(z_hbm.at[sidx_all.at[pl.ds(cb, C)]], srows, sa)
            pltpu.async_copy(z_hbm.at[didx_all.at[pl.ds(cb, C)]], drows, sb)

        def waitb(ci, b):
            srows, drows, sa, sb = b
            cb = ci * C
            pltpu.make_async_copy(
                z_hbm.at[sidx_all.at[pl.ds(cb, C)]], srows, sa).wait()
            pltpu.make_async_copy(
                z_hbm.at[didx_all.at[pl.ds(cb, C)]], drows, sb).wait()

        ii = lax.iota(jnp.int32, L)

        def compute(ci, b):
            srows, drows, _, _ = b
            off = base + ci * C

            def group(g, carry2):
                base_e = g * L

                def edge(j, svec):
                    e = base_e + j
                    parts = []
                    for m in range(4):
                        pr = (srows[e, pl.ds(m * 64, 2 * L)]
                              * drows[e, pl.ds(m * 64, 2 * L)])
                        pa, pb = plsc.unpack(pr,
                                             format=plsc.PackFormat.INTERLEAVED)
                        p0 = pa + pb
                        pr = (srows[e, pl.ds(m * 64 + 2 * L, 2 * L)]
                              * drows[e, pl.ds(m * 64 + 2 * L, 2 * L)])
                        pa, pb = plsc.unpack(pr,
                                             format=plsc.PackFormat.INTERLEAVED)
                        parts.append(p0 + (pa + pb))
                    acc = (parts[0] + parts[1]) + (parts[2] + parts[3])
                    return jnp.where(ii == j, jnp.sum(acc), svec)

                svec = lax.fori_loop(0, L, edge, jnp.zeros((L,), jnp.float32),
                                     unroll=4)
                scv[pl.ds(base_e, L)] = svec
                return carry2

            lax.fori_loop(0, C // L, group, 0)
            pltpu.sync_copy(scv, out_hbm.at[pl.ds(off, C)])

        start(0, bufs[0])

        def body(p, carry):
            c0 = 2 * p
            start(c0 + 1, bufs[1])
            waitb(c0, bufs[0])
            compute(c0, bufs[0])

            @pl.when(c0 + 2 < NCHUNK)
            def _():
                start(c0 + 2, bufs[0])

            waitb(c0 + 1, bufs[1])
            compute(c0 + 1, bufs[1])
            return carry

        lax.fori_loop(0, NCHUNK // 2, body, 0)

    return score_kernel


def _loss_body(T, E, p_ref, n_ref, o_ref):
    p = p_ref[...]
    n = n_ref[...]
    sp = 1.0 / (1.0 + jnp.exp(-p))
    sn = 1.0 / (1.0 + jnp.exp(-n))
    tp = jnp.log(sp + EPS)
    tn = jnp.log(1.0 - sn + EPS)
    o_ref[0, 0] = -(jnp.sum(tp) + jnp.sum(tn)) / (T * E)


def kernel(ps, ns, zs):
    T, N, D = zs.shape
    E = ps.shape[2]

    zf = zs.astype(jnp.bfloat16).reshape(T * N, D)
    offs = (jnp.arange(T, dtype=jnp.int32) * N)[:, None, None]
    src = (jnp.stack([ps[:, 0, :], ns[:, 0, :]], axis=1).astype(jnp.int32)
           + offs).reshape(-1)
    dst = (jnp.stack([ps[:, 1, :], ns[:, 1, :]], axis=1).astype(jnp.int32)
           + offs).reshape(-1)

    scores = _make_score_kernel(T, N, D, E)(zf, src, dst)
    sc4 = scores.reshape(T, 2, E)
    pos = sc4[:, 0, :]
    neg = sc4[:, 1, :]

    loss = pl.pallas_call(
        functools.partial(_loss_body, T, E),
        out_shape=jax.ShapeDtypeStruct((1, 1), jnp.float32),
        in_specs=[
            pl.BlockSpec(memory_space=pltpu.VMEM),
            pl.BlockSpec(memory_space=pltpu.VMEM),
        ],
        out_specs=pl.BlockSpec(memory_space=pltpu.SMEM),
    )(pos, neg)
    return loss.reshape(1)


# speedup vs baseline: 17.0612x; 1.2141x over previous
import functools

import jax
import jax.numpy as jnp
from jax import lax
from jax.experimental import pallas as pl
from jax.experimental.pallas import tpu as pltpu
from jax.experimental.pallas import tpu_sc as plsc

NC, NS, L = 2, 16, 16
NW = NC * NS

EPS = 1e-6


def _make_score_kernel(T, N, D, E):
    TOT = T * 2 * E
    PER_W = TOT // NW
    C = 80
    NCHUNK = PER_W // C
    assert PER_W * NW == TOT and NCHUNK * C == PER_W and D == 16 * L

    mesh = plsc.VectorSubcoreMesh(core_axis_name="c", subcore_axis_name="s")

    @functools.partial(
        pl.kernel,
        mesh=mesh,
        out_type=jax.ShapeDtypeStruct((TOT,), jnp.float32),
        compiler_params=pltpu.CompilerParams(use_tc_tiling_on_sc=False,
                                             needs_layout_passes=False),
        scratch_types=[
            [pltpu.VMEM((C,), jnp.int32) for _ in range(4)],
            [pltpu.VMEM((C,), jnp.int32) for _ in range(4)],
            [pltpu.VMEM((C, D), jnp.bfloat16) for _ in range(4)],
            [pltpu.VMEM((C, D), jnp.bfloat16) for _ in range(4)],
            pltpu.VMEM((C,), jnp.float32),
            [pltpu.SemaphoreType.DMA for _ in range(4)],
            [pltpu.SemaphoreType.DMA for _ in range(4)],
            [pltpu.SemaphoreType.DMA for _ in range(4)],
            [pltpu.SemaphoreType.DMA for _ in range(4)],
        ],
    )
    def score_kernel(z_hbm, src_hbm, dst_hbm, out_hbm,
                     SI, DI, SR, DR, scv, semsi, semdi, semsr, semdr):
        wid = lax.axis_index("s") * NC + lax.axis_index("c")
        base = wid * PER_W

        def idx_start(ci, k):
            off = base + ci * C
            pltpu.async_copy(src_hbm.at[pl.ds(off, C)], SI[k], semsi[k])
            pltpu.async_copy(dst_hbm.at[pl.ds(off, C)], DI[k], semdi[k])

        def idx_wait(ci, k):
            off = base + ci * C
            pltpu.make_async_copy(src_hbm.at[pl.ds(off, C)], SI[k],
                                  semsi[k]).wait()
            pltpu.make_async_copy(dst_hbm.at[pl.ds(off, C)], DI[k],
                                  semdi[k]).wait()

        def row_start(ci, k):
            pltpu.async_copy(z_hbm.at[SI[k]], SR[k], semsr[k])
            pltpu.async_copy(z_hbm.at[DI[k]], DR[k], semdr[k])

        def row_wait(ci, k):
            pltpu.make_async_copy(z_hbm.at[SI[k]], SR[k], semsr[k]).wait()
            pltpu.make_async_copy(z_hbm.at[DI[k]], DR[k], semdr[k]).wait()

        ii = lax.iota(jnp.int32, L)

        def compute(ci, k):
            srows, drows = SR[k], DR[k]
            off = base + ci * C

            def group(g, carry2):
                base_e = g * L

                def edge(j, svec):
                    e = base_e + j
                    parts = []
                    for m in range(4):
                        pr = (srows[e, pl.ds(m * 64, 2 * L)]
                              * drows[e, pl.ds(m * 64, 2 * L)])
                        pa, pb = plsc.unpack(pr,
                                             format=plsc.PackFormat.INTERLEAVED)
                        p0 = pa + pb
                        pr = (srows[e, pl.ds(m * 64 + 2 * L, 2 * L)]
                              * drows[e, pl.ds(m * 64 + 2 * L, 2 * L)])
                        pa, pb = plsc.unpack(pr,
                                             format=plsc.PackFormat.INTERLEAVED)
                        parts.append(p0 + (pa + pb))
                    acc = (parts[0] + parts[1]) + (parts[2] + parts[3])
                    return jnp.where(ii == j, jnp.sum(acc), svec)

                svec = lax.fori_loop(0, L, edge, jnp.zeros((L,), jnp.float32),
                                     unroll=4)
                scv[pl.ds(base_e, L)] = svec
                return carry2

            lax.fori_loop(0, C // L, group, 0)
            pltpu.sync_copy(scv, out_hbm.at[pl.ds(off, C)])

        for k in range(3):
            idx_start(k, k)
        for k in range(3):
            idx_wait(k, k)
            row_start(k, k)
        idx_start(3, 3)

        def body(q, carry):
            c0 = 4 * q
            for ph in range(4):
                c = c0 + ph
                row_wait(c, ph)

                @pl.when(c + 4 < NCHUNK)
                def _():
                    idx_start(c + 4, ph)

                @pl.when(c + 3 < NCHUNK)
                def _():
                    idx_wait(c + 3, (ph + 3) % 4)
                    row_start(c + 3, (ph + 3) % 4)

                compute(c, ph)
            return carry

        lax.fori_loop(0, NCHUNK // 4, body, 0)

    return score_kernel


def _loss_body(T, E, p_ref, n_ref, o_ref):
    p = p_ref[...]
    n = n_ref[...]
    sp = 1.0 / (1.0 + jnp.exp(-p))
    sn = 1.0 / (1.0 + jnp.exp(-n))
    tp = jnp.log(sp + EPS)
    tn = jnp.log(1.0 - sn + EPS)
    o_ref[0, 0] = -(jnp.sum(tp) + jnp.sum(tn)) / (T * E)


def kernel(ps, ns, zs):
    T, N, D = zs.shape
    E = ps.shape[2]

    zf = zs.astype(jnp.bfloat16).reshape(T * N, D)
    offs = (jnp.arange(T, dtype=jnp.int32) * N)[:, None, None]
    src = (jnp.stack([ps[:, 0, :], ns[:, 0, :]], axis=1).astype(jnp.int32)
           + offs).reshape(-1)
    dst = (jnp.stack([ps[:, 1, :], ns[:, 1, :]], axis=1).astype(jnp.int32)
           + offs).reshape(-1)

    scores = _make_score_kernel(T, N, D, E)(zf, src, dst)
    sc4 = scores.reshape(T, 2, E)
    pos = sc4[:, 0, :]
    neg = sc4[:, 1, :]

    loss = pl.pallas_call(
        functools.partial(_loss_body, T, E),
        out_shape=jax.ShapeDtypeStruct((1, 1), jnp.float32),
        in_specs=[
            pl.BlockSpec(memory_space=pltpu.VMEM),
            pl.BlockSpec(memory_space=pltpu.VMEM),
        ],
        out_specs=pl.BlockSpec(memory_space=pltpu.SMEM),
    )(pos, neg)
    return loss.reshape(1)
